# baseline (speedup 1.0000x reference)
import jax
import jax.numpy as jnp
import numpy as np
from jax import lax
from jax.experimental import pallas as pl
from jax.experimental.pallas import tpu as pltpu
from jax.experimental.pallas import tpu_sc as plsc

NC = 2
NS = 16
NW = NC * NS
LANES = 16

B = 16384
L = 200
D = 64
HALF = L // 2
NCATE = 1024

ROWS_PER_W = B // NW
G = 4
NGRP = ROWS_PER_W // G
TGT_CHUNK = 32
NTC = ROWS_PER_W // TGT_CHUNK

MASK_HI = np.int32(-65536)

PERM_IH = np.concatenate([
    np.arange(0, 32, 2), np.arange(1, 32, 2),
    np.arange(32, 64, 2), np.arange(33, 64, 2),
])
_j = np.arange(16)
PERM_CNT = (
    np.arange(32)[:, None] * 32
    + np.stack([_j, _j + 16], axis=1).reshape(32)[None, :]
).reshape(1024)


def _sc_gather_kernel(item_hist_hbm, cate_hist_hbm, titem_hbm, tcate_hbm,
                      item_tab_hbm, cate_tab_hbm,
                      ti_out, tc_out, ihs_out, cnt_out,
                      ih_idx, ch_idx, ibuf, obuf_i, cbuf, cbufb,
                      tidx_i, tidx_c, tbuf,
                      gsem, isem, ssem, tgsem, tssem):
  cid = lax.axis_index("c")
  sid = lax.axis_index("s")
  wid = sid * NC + cid
  base = wid * ROWS_PER_W

  pltpu.sync_copy(titem_hbm.at[pl.ds(base, ROWS_PER_W)], tidx_i)
  pltpu.sync_copy(tcate_hbm.at[pl.ds(base, ROWS_PER_W)], tidx_c)

  def tgt_gather(c):
    tab = item_tab_hbm if c < NTC else cate_tab_hbm
    idx = tidx_i if c < NTC else tidx_c
    off = (c % NTC) * TGT_CHUNK
    return pltpu.make_async_copy(
        tab.at[idx.at[pl.ds(off, TGT_CHUNK)]], tbuf.at[c % 4],
        tgsem.at[c % 4])

  def tgt_store(c):
    dst = ti_out if c < NTC else tc_out
    off = base + (c % NTC) * TGT_CHUNK
    return pltpu.make_async_copy(
        tbuf.at[c % 4], dst.at[pl.ds(off, TGT_CHUNK)], tssem.at[c % 4])

  tgt_gather(0).start()
  tgt_gather(1).start()
  for c in range(2 * NTC):
    tgt_gather(c).wait()
    if c >= 2:
      tgt_store(c - 2).wait()
    tgt_store(c).start()
    if c + 2 < 2 * NTC:
      tgt_gather(c + 2).start()
  tgt_store(2 * NTC - 2).wait()
  tgt_store(2 * NTC - 1).wait()

  ones = jnp.ones((LANES,), jnp.float32)
  lastmask = lax.iota(jnp.int32, LANES) >= (2 * LANES - (L % (2 * LANES)))
  zeros = jnp.zeros((LANES,), jnp.float32)

  def idx_copies(k, p):
    b0 = base + k * G
    return [
        pltpu.make_async_copy(
            item_hist_hbm.at[pl.ds(b0, G)], ih_idx.at[p], isem.at[p]),
        pltpu.make_async_copy(
            cate_hist_hbm.at[pl.ds(b0, G)], ch_idx.at[p], isem.at[p]),
    ]

  def gather_copies(p):
    cps = []
    for g in range(G):
      for h in range(2):
        cps.append(pltpu.make_async_copy(
            item_tab_hbm.at[ih_idx.at[p, g, h]], ibuf.at[p, g, h],
            gsem.at[p]))
    return cps

  def store_copies(k, p):
    b0 = base + k * G
    return [
        pltpu.make_async_copy(
            obuf_i.at[p], ihs_out.at[pl.ds(b0, G)], ssem.at[p]),
        pltpu.make_async_copy(
            cbufb.at[p], cnt_out.at[pl.ds(b0, G)], ssem.at[p]),
    ]

  for p in range(2):
    for c in idx_copies(p, p):
      c.start()
      c.wait()
    for c in gather_copies(p):
      c.start()

  def loop_body(jj, carry):
    for m in range(2):
      p = m
      k = 2 * jj + m
      for c in gather_copies(p):
        c.wait()
      @pl.when(jj >= 1)
      def _():
        for c in store_copies(k, p):
          c.wait()
      for g in range(G):
        for c in range(NCATE // LANES):
          cbuf[p, g, pl.ds(c * LANES, LANES)] = zeros
        gsplat = jnp.full((LANES,), g, jnp.int32)
        for c in range(L // LANES):
          idxv = ch_idx[p, g, pl.ds(c * LANES, LANES)]
          plsc.addupdate_scatter(cbuf.at[p], [gsplat, idxv], ones)
        idxv = ch_idx[p, g, pl.ds(L - LANES, LANES)]
        plsc.addupdate_scatter(cbuf.at[p], [gsplat, idxv], ones,
                               mask=lastmask)
        for kk in range(NCATE // 32):
          av = plsc.bitcast(cbuf[p, g, pl.ds(32 * kk, LANES)], jnp.int32)
          bv = plsc.bitcast(cbuf[p, g, pl.ds(32 * kk + 16, LANES)],
                            jnp.int32)
          packed = lax.shift_right_logical(av, 16) | (bv & MASK_HI)
          cbufb[p, g, pl.ds(32 * kk, 32)] = plsc.bitcast(
              packed, jnp.bfloat16)
      @pl.when(jj <= (NGRP // 2 - 2))
      def _():
        for c in idx_copies(k + 2, p):
          c.start()
      for g in range(G):
        def red_body(i, accs):
          l0, h0, l1, h1 = accs
          for h in range(2):
            for u in range(4):
              r = i * 4 + u
              x0 = plsc.bitcast(ibuf[p, g, h, r, pl.ds(0, 32)], jnp.int32)
              x1 = plsc.bitcast(ibuf[p, g, h, r, pl.ds(32, 32)], jnp.int32)
              l0 = l0 + plsc.bitcast(lax.shift_left(x0, 16), jnp.float32)
              h0 = h0 + plsc.bitcast(x0 & MASK_HI, jnp.float32)
              l1 = l1 + plsc.bitcast(lax.shift_left(x1, 16), jnp.float32)
              h1 = h1 + plsc.bitcast(x1 & MASK_HI, jnp.float32)
          return l0, h0, l1, h1

        accs = lax.fori_loop(0, HALF // 4, red_body, (zeros,) * 4)
        for d in range(4):
          obuf_i[p, g, pl.ds(d * LANES, LANES)] = accs[d]
      for c in store_copies(k, p):
        c.start()
      @pl.when(jj <= (NGRP // 2 - 2))
      def _():
        for c in idx_copies(k + 2, p):
          c.wait()
        for c in gather_copies(p):
          c.start()
    return carry

  lax.fori_loop(0, NGRP // 2, loop_body, 0)

  for p in range(2):
    for c in store_copies(NGRP - 2 + p, p):
      c.wait()


def _sc_gather(item_hist, cate_hist, titem, tcate, item_tab, cate_tab):
  mesh = plsc.VectorSubcoreMesh(core_axis_name="c", subcore_axis_name="s",
                                num_cores=NC, num_subcores=NS)
  f32 = jnp.float32
  i32 = jnp.int32
  bf16 = jnp.bfloat16
  out_type = (
      jax.ShapeDtypeStruct((B, D), bf16),
      jax.ShapeDtypeStruct((B, D), bf16),
      jax.ShapeDtypeStruct((B, D), f32),
      jax.ShapeDtypeStruct((B, NCATE), bf16),
  )
  scratch = [
      pltpu.VMEM((2, G, 2, HALF), i32),
      pltpu.VMEM((2, G, L), i32),
      pltpu.VMEM((2, G, 2, HALF, D), bf16),
      pltpu.VMEM((2, G, D), f32),
      pltpu.VMEM((2, G, NCATE), f32),
      pltpu.VMEM((2, G, NCATE), bf16),
      pltpu.VMEM((ROWS_PER_W,), i32),
      pltpu.VMEM((ROWS_PER_W,), i32),
      pltpu.VMEM((4, TGT_CHUNK, D), bf16),
      pltpu.SemaphoreType.DMA((2,)),
      pltpu.SemaphoreType.DMA((2,)),
      pltpu.SemaphoreType.DMA((2,)),
      pltpu.SemaphoreType.DMA((4,)),
      pltpu.SemaphoreType.DMA((4,)),
  ]
  fn = pl.kernel(_sc_gather_kernel, out_type=out_type, mesh=mesh,
                 scratch_types=scratch,
                 compiler_params=pltpu.CompilerParams(
                     use_tc_tiling_on_sc=False,
                     needs_layout_passes=False))
  return fn(item_hist, cate_hist, titem, tcate, item_tab, cate_tab)


BS = 1024


def _mlp_kernel(ti, tc, ihs, cnt, ctab, w1, b1, a1, w2, b2, a2, w3, b3, out):
  f32 = jnp.float32
  chs = jnp.dot(cnt[...], ctab[...], preferred_element_type=f32)
  x = jnp.concatenate(
      [ti[...].astype(f32), tc[...].astype(f32), ihs[...], chs], axis=1)
  h1 = jnp.dot(x, w1[...], preferred_element_type=f32) + b1[...]
  h1 = jnp.maximum(h1, 0.0) + a1[...] * jnp.minimum(h1, 0.0)
  h2 = jnp.dot(h1, w2[...], preferred_element_type=f32) + b2[...]
  h2 = jnp.maximum(h2, 0.0) + a2[...] * jnp.minimum(h2, 0.0)
  z = jnp.dot(h2, w3[...], preferred_element_type=f32) + b3[...]
  m = jnp.max(z, axis=-1, keepdims=True)
  e = jnp.exp(z - m)
  out[...] = e / jnp.sum(e, axis=-1, keepdims=True) + 1e-8


def _mlp(ti, tc, ihs, cnt, ctab, w1, b1, a1, w2, b2, a2, w3, b3):
  n1, n2 = w2.shape
  n3 = w3.shape[1]
  row = lambda i: (i, 0)
  full = lambda i: (0, 0)
  bspec = lambda w: pl.BlockSpec((BS, w), row)
  wspec = lambda s: pl.BlockSpec(s, full)
  return pl.pallas_call(
      _mlp_kernel,
      grid=(B // BS,),
      in_specs=[
          bspec(D), bspec(D), bspec(D), bspec(NCATE),
          wspec((NCATE, D)),
          wspec((4 * D, n1)), wspec((1, n1)), wspec((1, n1)),
          wspec((n1, n2)), wspec((1, n2)), wspec((1, n2)),
          wspec((n2, n3)), wspec((1, n3)),
      ],
      out_specs=pl.BlockSpec((BS, n3), row),
      out_shape=jax.ShapeDtypeStruct((B, n3), jnp.float32),
  )(ti, tc, ihs, cnt, ctab, w1, b1, a1, w2, b2, a2, w3, b3)


def kernel(item_history, cate_history, targetitem, targetcate, item_lookup,
           cate_lookup, gamma, beta, W1, b1, a1, W2, b2, a2, W3, b3):
  bf16 = jnp.bfloat16
  tab16 = item_lookup.astype(bf16)
  ctab16 = cate_lookup.astype(bf16)
  ih = item_history.reshape(B, 2, HALF)
  ti, tc, ihs, cnt = _sc_gather(ih, cate_history, targetitem, targetcate,
                                tab16, ctab16)
  ctab = jnp.pad(cate_lookup, ((0, NCATE - cate_lookup.shape[0]), (0, 0)))
  ctab_p = ctab[PERM_CNT].astype(bf16)
  scale = gamma * (1.0 / jnp.sqrt(1.0 + 1e-3))
  w1 = W1 * scale[:, None]
  w1 = jnp.concatenate([w1[:2 * D], w1[2 * D:3 * D][PERM_IH], w1[3 * D:]])
  b1e = (b1 + beta @ W1).reshape(1, -1)
  out = _mlp(ti, tc, ihs, cnt, ctab_p, w1, b1e, a1.reshape(1, -1),
             W2, b2.reshape(1, -1), a2.reshape(1, -1),
             W3, b3.reshape(1, -1))
  return out


# speedup vs baseline: 1.1184x; 1.1184x over previous
import jax
import jax.numpy as jnp
import numpy as np
from jax import lax
from jax.experimental import pallas as pl
from jax.experimental.pallas import tpu as pltpu
from jax.experimental.pallas import tpu_sc as plsc

NC = 2
NS = 16
NW = NC * NS
LANES = 16

B = 16384
L = 200
D = 64
HALF = L // 2
NCATE = 1024

ROWS_PER_W = B // NW
G = 4
NGRP = ROWS_PER_W // G
TGT_CHUNK = 32
NTC = ROWS_PER_W // TGT_CHUNK

MASK_HI = np.int32(-65536)

_j = np.arange(16)
PERM_CNT = (
    np.arange(32)[:, None] * 32
    + np.stack([_j, _j + 16], axis=1).reshape(32)[None, :]
).reshape(1024)


def _sc_gather_kernel(item_hist_hbm, cate_hist_hbm, titem_hbm, tcate_hbm,
                      item_tab_hbm, cate_tab_hbm,
                      ti_out, tc_out, ihs_out, cnt_out,
                      ih_idx, ch_idx, ibuf, obuf_i, cbuf, cbufb,
                      tidx_i, tidx_c, tbuf,
                      gsem, isem, ssem, tgsem, tssem):
  cid = lax.axis_index("c")
  sid = lax.axis_index("s")
  wid = sid * NC + cid
  base = wid * ROWS_PER_W

  pltpu.sync_copy(titem_hbm.at[pl.ds(base, ROWS_PER_W)], tidx_i)
  pltpu.sync_copy(tcate_hbm.at[pl.ds(base, ROWS_PER_W)], tidx_c)

  def tgt_gather(c):
    tab = item_tab_hbm if c < NTC else cate_tab_hbm
    idx = tidx_i if c < NTC else tidx_c
    off = (c % NTC) * TGT_CHUNK
    return pltpu.make_async_copy(
        tab.at[idx.at[pl.ds(off, TGT_CHUNK)]], tbuf.at[c % 4],
        tgsem.at[c % 4])

  def tgt_store(c):
    dst = ti_out if c < NTC else tc_out
    off = base + (c % NTC) * TGT_CHUNK
    return pltpu.make_async_copy(
        tbuf.at[c % 4], dst.at[pl.ds(off, TGT_CHUNK)], tssem.at[c % 4])

  tgt_gather(0).start()
  tgt_gather(1).start()
  for c in range(2 * NTC):
    tgt_gather(c).wait()
    if c >= 2:
      tgt_store(c - 2).wait()
    tgt_store(c).start()
    if c + 2 < 2 * NTC:
      tgt_gather(c + 2).start()
  tgt_store(2 * NTC - 2).wait()
  tgt_store(2 * NTC - 1).wait()

  ones = jnp.ones((LANES,), jnp.float32)
  lastmask = lax.iota(jnp.int32, LANES) >= (2 * LANES - (L % (2 * LANES)))
  zeros = jnp.zeros((LANES,), jnp.float32)

  def idx_copies(k, p):
    b0 = base + k * G
    return [
        pltpu.make_async_copy(
            item_hist_hbm.at[pl.ds(b0, G)], ih_idx.at[p], isem.at[p]),
        pltpu.make_async_copy(
            cate_hist_hbm.at[pl.ds(b0, G)], ch_idx.at[p], isem.at[p]),
    ]

  def gather_copies(p):
    cps = []
    for g in range(G):
      for h in range(2):
        cps.append(pltpu.make_async_copy(
            item_tab_hbm.at[ih_idx.at[p, g, h]], ibuf.at[p, g, h],
            gsem.at[p]))
    return cps

  def store_copies(k, p):
    b0 = base + k * G
    return [
        pltpu.make_async_copy(
            obuf_i.at[p], ihs_out.at[pl.ds(b0, G)], ssem.at[p]),
        pltpu.make_async_copy(
            cbufb.at[p], cnt_out.at[pl.ds(b0, G)], ssem.at[p]),
    ]

  for p in range(2):
    for c in idx_copies(p, p):
      c.start()
      c.wait()
    for c in gather_copies(p):
      c.start()

  def loop_body(jj, carry):
    for m in range(2):
      p = m
      k = 2 * jj + m
      for c in gather_copies(p):
        c.wait()
      @pl.when(jj >= 1)
      def _():
        for c in store_copies(k, p):
          c.wait()
      for g in range(G):
        for c in range(NCATE // LANES):
          cbuf[p, g, pl.ds(c * LANES, LANES)] = zeros
        gsplat = jnp.full((LANES,), g, jnp.int32)
        for c in range(L // LANES):
          idxv = ch_idx[p, g, pl.ds(c * LANES, LANES)]
          plsc.addupdate_scatter(cbuf.at[p], [gsplat, idxv], ones)
        idxv = ch_idx[p, g, pl.ds(L - LANES, LANES)]
        plsc.addupdate_scatter(cbuf.at[p], [gsplat, idxv], ones,
                               mask=lastmask)
        for kk in range(NCATE // 32):
          av = plsc.bitcast(cbuf[p, g, pl.ds(32 * kk, LANES)], jnp.int32)
          bv = plsc.bitcast(cbuf[p, g, pl.ds(32 * kk + 16, LANES)],
                            jnp.int32)
          packed = lax.shift_right_logical(av, 16) | (bv & MASK_HI)
          cbufb[p, g, pl.ds(32 * kk, 32)] = plsc.bitcast(
              packed, jnp.bfloat16)
      @pl.when(jj <= (NGRP // 2 - 2))
      def _():
        for c in idx_copies(k + 2, p):
          c.start()
      for g in range(G):
        def red_body(i, accs):
          a0, a1, a2, a3 = accs
          for h in range(2):
            for u in range(4):
              r = i * 4 + u
              a0 = a0 + ibuf[p, g, h, r, pl.ds(0, LANES)]
              a1 = a1 + ibuf[p, g, h, r, pl.ds(16, LANES)]
              a2 = a2 + ibuf[p, g, h, r, pl.ds(32, LANES)]
              a3 = a3 + ibuf[p, g, h, r, pl.ds(48, LANES)]
          return a0, a1, a2, a3

        accs = lax.fori_loop(0, HALF // 4, red_body, (zeros,) * 4)
        for d in range(4):
          obuf_i[p, g, pl.ds(d * LANES, LANES)] = accs[d]
      for c in store_copies(k, p):
        c.start()
      @pl.when(jj <= (NGRP // 2 - 2))
      def _():
        for c in idx_copies(k + 2, p):
          c.wait()
        for c in gather_copies(p):
          c.start()
    return carry

  lax.fori_loop(0, NGRP // 2, loop_body, 0)

  for p in range(2):
    for c in store_copies(NGRP - 2 + p, p):
      c.wait()


def _sc_gather(item_hist, cate_hist, titem, tcate, item_tab, cate_tab):
  mesh = plsc.VectorSubcoreMesh(core_axis_name="c", subcore_axis_name="s",
                                num_cores=NC, num_subcores=NS)
  f32 = jnp.float32
  i32 = jnp.int32
  out_type = (
      jax.ShapeDtypeStruct((B, D), f32),
      jax.ShapeDtypeStruct((B, D), f32),
      jax.ShapeDtypeStruct((B, D), f32),
      jax.ShapeDtypeStruct((B, NCATE), jnp.bfloat16),
  )
  scratch = [
      pltpu.VMEM((2, G, 2, HALF), i32),
      pltpu.VMEM((2, G, L), i32),
      pltpu.VMEM((2, G, 2, HALF, D), f32),
      pltpu.VMEM((2, G, D), f32),
      pltpu.VMEM((2, G, NCATE), f32),
      pltpu.VMEM((2, G, NCATE), jnp.bfloat16),
      pltpu.VMEM((ROWS_PER_W,), i32),
      pltpu.VMEM((ROWS_PER_W,), i32),
      pltpu.VMEM((4, TGT_CHUNK, D), f32),
      pltpu.SemaphoreType.DMA((2,)),
      pltpu.SemaphoreType.DMA((2,)),
      pltpu.SemaphoreType.DMA((2,)),
      pltpu.SemaphoreType.DMA((4,)),
      pltpu.SemaphoreType.DMA((4,)),
  ]
  fn = pl.kernel(_sc_gather_kernel, out_type=out_type, mesh=mesh,
                 scratch_types=scratch,
                 compiler_params=pltpu.CompilerParams(
                     use_tc_tiling_on_sc=False,
                     needs_layout_passes=False))
  return fn(item_hist, cate_hist, titem, tcate, item_tab, cate_tab)


BS = 1024


def _mlp_kernel(ti, tc, ihs, cnt, ctab, w1, b1, a1, w2, b2, a2, w3, b3, out):
  f32 = jnp.float32
  chs = jnp.dot(cnt[...], ctab[...], preferred_element_type=f32)
  x = jnp.concatenate([ti[...], tc[...], ihs[...], chs], axis=1)
  h1 = jnp.dot(x, w1[...], preferred_element_type=f32) + b1[...]
  h1 = jnp.maximum(h1, 0.0) + a1[...] * jnp.minimum(h1, 0.0)
  h2 = jnp.dot(h1, w2[...], preferred_element_type=f32) + b2[...]
  h2 = jnp.maximum(h2, 0.0) + a2[...] * jnp.minimum(h2, 0.0)
  z = jnp.dot(h2, w3[...], preferred_element_type=f32) + b3[...]
  m = jnp.max(z, axis=-1, keepdims=True)
  e = jnp.exp(z - m)
  out[...] = e / jnp.sum(e, axis=-1, keepdims=True) + 1e-8


def _mlp(ti, tc, ihs, cnt, ctab, w1, b1, a1, w2, b2, a2, w3, b3):
  n1, n2 = w2.shape
  n3 = w3.shape[1]
  row = lambda i: (i, 0)
  full = lambda i: (0, 0)
  bspec = lambda w: pl.BlockSpec((BS, w), row)
  wspec = lambda s: pl.BlockSpec(s, full)
  return pl.pallas_call(
      _mlp_kernel,
      grid=(B // BS,),
      in_specs=[
          bspec(D), bspec(D), bspec(D), bspec(NCATE),
          wspec((NCATE, D)),
          wspec((4 * D, n1)), wspec((1, n1)), wspec((1, n1)),
          wspec((n1, n2)), wspec((1, n2)), wspec((1, n2)),
          wspec((n2, n3)), wspec((1, n3)),
      ],
      out_specs=pl.BlockSpec((BS, n3), row),
      out_shape=jax.ShapeDtypeStruct((B, n3), jnp.float32),
  )(ti, tc, ihs, cnt, ctab, w1, b1, a1, w2, b2, a2, w3, b3)


def kernel(item_history, cate_history, targetitem, targetcate, item_lookup,
           cate_lookup, gamma, beta, W1, b1, a1, W2, b2, a2, W3, b3):
  ih = item_history.reshape(B, 2, HALF)
  ti, tc, ihs, cnt = _sc_gather(ih, cate_history, targetitem, targetcate,
                                item_lookup, cate_lookup)
  ctab = jnp.pad(cate_lookup, ((0, NCATE - cate_lookup.shape[0]), (0, 0)))
  ctab_p = ctab[PERM_CNT].astype(jnp.bfloat16)
  scale = gamma * (1.0 / jnp.sqrt(1.0 + 1e-3))
  w1 = W1 * scale[:, None]
  b1e = (b1 + beta @ W1).reshape(1, -1)
  out = _mlp(ti, tc, ihs, cnt, ctab_p, w1, b1e, a1.reshape(1, -1),
             W2, b2.reshape(1, -1), a2.reshape(1, -1),
             W3, b3.reshape(1, -1))
  return out


# speedup vs baseline: 1.2157x; 1.0871x over previous
import jax
import jax.numpy as jnp
import numpy as np
from jax import lax
from jax.experimental import pallas as pl
from jax.experimental.pallas import tpu as pltpu
from jax.experimental.pallas import tpu_sc as plsc

NC = 2
NS = 16
NW = NC * NS
LANES = 16

B = 16384
L = 200
D = 64
HALF = L // 2
NCATE = 1024

ROWS_PER_W = B // NW
G = 4
NGRP = ROWS_PER_W // G
TGT_CHUNK = 32
NTC = ROWS_PER_W // TGT_CHUNK


def _sc_gather_kernel(item_hist_hbm, cate_hist_hbm, titem_hbm, tcate_hbm,
                      item_tab_hbm, cate_tab_hbm,
                      ti_out, tc_out, ihs_out, cnt_out,
                      ih_idx, ch_idx, ibuf, obuf_i, cbuf,
                      tidx_i, tidx_c, tbuf,
                      gsem, isem, ssem, tgsem, tssem):
  cid = lax.axis_index("c")
  sid = lax.axis_index("s")
  wid = sid * NC + cid
  base = wid * ROWS_PER_W

  pltpu.sync_copy(titem_hbm.at[pl.ds(base, ROWS_PER_W)], tidx_i)
  pltpu.sync_copy(tcate_hbm.at[pl.ds(base, ROWS_PER_W)], tidx_c)

  def tgt_gather(c):
    tab = item_tab_hbm if c < NTC else cate_tab_hbm
    idx = tidx_i if c < NTC else tidx_c
    off = (c % NTC) * TGT_CHUNK
    return pltpu.make_async_copy(
        tab.at[idx.at[pl.ds(off, TGT_CHUNK)]], tbuf.at[c % 4],
        tgsem.at[c % 4])

  def tgt_store(c):
    dst = ti_out if c < NTC else tc_out
    off = base + (c % NTC) * TGT_CHUNK
    return pltpu.make_async_copy(
        tbuf.at[c % 4], dst.at[pl.ds(off, TGT_CHUNK)], tssem.at[c % 4])

  tgt_gather(0).start()
  tgt_gather(1).start()
  for c in range(2 * NTC):
    tgt_gather(c).wait()
    if c >= 2:
      tgt_store(c - 2).wait()
    tgt_store(c).start()
    if c + 2 < 2 * NTC:
      tgt_gather(c + 2).start()
  tgt_store(2 * NTC - 2).wait()
  tgt_store(2 * NTC - 1).wait()

  ones = jnp.ones((LANES,), jnp.float32)
  lastmask = lax.iota(jnp.int32, LANES) >= (2 * LANES - (L % (2 * LANES)))
  zeros = jnp.zeros((LANES,), jnp.float32)

  def idx_copies(k, p):
    b0 = base + k * G
    return [
        pltpu.make_async_copy(
            item_hist_hbm.at[pl.ds(b0, G)], ih_idx.at[p], isem.at[p]),
        pltpu.make_async_copy(
            cate_hist_hbm.at[pl.ds(b0, G)], ch_idx.at[p], isem.at[p]),
    ]

  def gather_copies(p):
    cps = []
    for g in range(G):
      for h in range(2):
        cps.append(pltpu.make_async_copy(
            item_tab_hbm.at[ih_idx.at[p, g, h]], ibuf.at[p, g, h],
            gsem.at[p]))
    return cps

  def store_copies(k, p):
    b0 = base + k * G
    return [
        pltpu.make_async_copy(
            obuf_i.at[p], ihs_out.at[pl.ds(b0, G)], ssem.at[p]),
        pltpu.make_async_copy(
            cbuf.at[p], cnt_out.at[pl.ds(b0, G)], ssem.at[p]),
    ]

  for p in range(2):
    for c in idx_copies(p, p):
      c.start()
      c.wait()
    for c in gather_copies(p):
      c.start()

  def loop_body(jj, carry):
    for m in range(2):
      p = m
      k = 2 * jj + m
      for c in gather_copies(p):
        c.wait()
      @pl.when(jj >= 1)
      def _():
        for c in store_copies(k, p):
          c.wait()
      for g in range(G):
        for c in range(NCATE // LANES):
          cbuf[p, g, pl.ds(c * LANES, LANES)] = zeros
        gsplat = jnp.full((LANES,), g, jnp.int32)
        for c in range(L // LANES):
          idxv = ch_idx[p, g, pl.ds(c * LANES, LANES)]
          plsc.addupdate_scatter(cbuf.at[p], [gsplat, idxv], ones)
        idxv = ch_idx[p, g, pl.ds(L - LANES, LANES)]
        plsc.addupdate_scatter(cbuf.at[p], [gsplat, idxv], ones,
                               mask=lastmask)
      @pl.when(jj <= (NGRP // 2 - 2))
      def _():
        for c in idx_copies(k + 2, p):
          c.start()
      for g in range(G):
        def red_body(i, accs):
          a0, a1, a2, a3 = accs
          for h in range(2):
            for u in range(4):
              r = i * 4 + u
              a0 = a0 + ibuf[p, g, h, r, pl.ds(0, LANES)]
              a1 = a1 + ibuf[p, g, h, r, pl.ds(16, LANES)]
              a2 = a2 + ibuf[p, g, h, r, pl.ds(32, LANES)]
              a3 = a3 + ibuf[p, g, h, r, pl.ds(48, LANES)]
          return a0, a1, a2, a3

        accs = lax.fori_loop(0, HALF // 4, red_body, (zeros,) * 4)
        for d in range(4):
          obuf_i[p, g, pl.ds(d * LANES, LANES)] = accs[d]
      for c in store_copies(k, p):
        c.start()
      @pl.when(jj <= (NGRP // 2 - 2))
      def _():
        for c in idx_copies(k + 2, p):
          c.wait()
        for c in gather_copies(p):
          c.start()
    return carry

  lax.fori_loop(0, NGRP // 2, loop_body, 0)

  for p in range(2):
    for c in store_copies(NGRP - 2 + p, p):
      c.wait()


def _sc_gather(item_hist, cate_hist, titem, tcate, item_tab, cate_tab):
  mesh = plsc.VectorSubcoreMesh(core_axis_name="c", subcore_axis_name="s",
                                num_cores=NC, num_subcores=NS)
  f32 = jnp.float32
  i32 = jnp.int32
  out_type = (
      jax.ShapeDtypeStruct((B, D), f32),
      jax.ShapeDtypeStruct((B, D), f32),
      jax.ShapeDtypeStruct((B, D), f32),
      jax.ShapeDtypeStruct((B, NCATE), f32),
  )
  scratch = [
      pltpu.VMEM((2, G, 2, HALF), i32),
      pltpu.VMEM((2, G, L), i32),
      pltpu.VMEM((2, G, 2, HALF, D), f32),
      pltpu.VMEM((2, G, D), f32),
      pltpu.VMEM((2, G, NCATE), f32),
      pltpu.VMEM((ROWS_PER_W,), i32),
      pltpu.VMEM((ROWS_PER_W,), i32),
      pltpu.VMEM((4, TGT_CHUNK, D), f32),
      pltpu.SemaphoreType.DMA((2,)),
      pltpu.SemaphoreType.DMA((2,)),
      pltpu.SemaphoreType.DMA((2,)),
      pltpu.SemaphoreType.DMA((4,)),
      pltpu.SemaphoreType.DMA((4,)),
  ]
  fn = pl.kernel(_sc_gather_kernel, out_type=out_type, mesh=mesh,
                 scratch_types=scratch,
                 compiler_params=pltpu.CompilerParams(
                     use_tc_tiling_on_sc=False,
                     needs_layout_passes=False))
  return fn(item_hist, cate_hist, titem, tcate, item_tab, cate_tab)


BS = 1024


def _mlp_kernel(ti, tc, ihs, cnt, ctab, w1, b1, a1, w2, b2, a2, w3, b3, out):
  f32 = jnp.float32
  chs = jnp.dot(cnt[...], ctab[...], preferred_element_type=f32)
  x = jnp.concatenate([ti[...], tc[...], ihs[...], chs], axis=1)
  h1 = jnp.dot(x, w1[...], preferred_element_type=f32) + b1[...]
  h1 = jnp.maximum(h1, 0.0) + a1[...] * jnp.minimum(h1, 0.0)
  h2 = jnp.dot(h1, w2[...], preferred_element_type=f32) + b2[...]
  h2 = jnp.maximum(h2, 0.0) + a2[...] * jnp.minimum(h2, 0.0)
  z = jnp.dot(h2, w3[...], preferred_element_type=f32) + b3[...]
  m = jnp.max(z, axis=-1, keepdims=True)
  e = jnp.exp(z - m)
  out[...] = e / jnp.sum(e, axis=-1, keepdims=True) + 1e-8


def _mlp(ti, tc, ihs, cnt, ctab, w1, b1, a1, w2, b2, a2, w3, b3):
  n1, n2 = w2.shape
  n3 = w3.shape[1]
  row = lambda i: (i, 0)
  full = lambda i: (0, 0)
  bspec = lambda w: pl.BlockSpec((BS, w), row)
  wspec = lambda s: pl.BlockSpec(s, full)
  return pl.pallas_call(
      _mlp_kernel,
      grid=(B // BS,),
      in_specs=[
          bspec(D), bspec(D), bspec(D), bspec(NCATE),
          wspec((NCATE, D)),
          wspec((4 * D, n1)), wspec((1, n1)), wspec((1, n1)),
          wspec((n1, n2)), wspec((1, n2)), wspec((1, n2)),
          wspec((n2, n3)), wspec((1, n3)),
      ],
      out_specs=pl.BlockSpec((BS, n3), row),
      out_shape=jax.ShapeDtypeStruct((B, n3), jnp.float32),
  )(ti, tc, ihs, cnt, ctab, w1, b1, a1, w2, b2, a2, w3, b3)


def kernel(item_history, cate_history, targetitem, targetcate, item_lookup,
           cate_lookup, gamma, beta, W1, b1, a1, W2, b2, a2, W3, b3):
  ih = item_history.reshape(B, 2, HALF)
  ti, tc, ihs, cnt = _sc_gather(ih, cate_history, targetitem, targetcate,
                                item_lookup, cate_lookup)
  ctab_p = jnp.pad(cate_lookup, ((0, NCATE - cate_lookup.shape[0]), (0, 0)))
  scale = gamma * (1.0 / jnp.sqrt(1.0 + 1e-3))
  w1 = W1 * scale[:, None]
  b1e = (b1 + beta @ W1).reshape(1, -1)
  out = _mlp(ti, tc, ihs, cnt, ctab_p, w1, b1e, a1.reshape(1, -1),
             W2, b2.reshape(1, -1), a2.reshape(1, -1),
             W3, b3.reshape(1, -1))
  return out


# speedup vs baseline: 1.2843x; 1.0564x over previous
import jax
import jax.numpy as jnp
import numpy as np
from jax import lax
from jax.experimental import pallas as pl
from jax.experimental.pallas import tpu as pltpu
from jax.experimental.pallas import tpu_sc as plsc

NC = 2
NS = 16
NW = NC * NS
LANES = 16

B = 16384
L = 200
D = 64
HALF = L // 2
NCATE = 1024

ROWS_PER_W = B // NW
G = 4
NGRP = ROWS_PER_W // G
TGT_CHUNK = 32
NTC = ROWS_PER_W // TGT_CHUNK


def _tgt_pipeline(tab_hbm, tidx, out_hbm, base, tbuf, tgsem, tssem):
  def tgt_gather(c):
    off = (c % NTC) * TGT_CHUNK
    return pltpu.make_async_copy(
        tab_hbm.at[tidx.at[pl.ds(off, TGT_CHUNK)]], tbuf.at[c % 4],
        tgsem.at[c % 4])

  def tgt_store(c):
    off = base + (c % NTC) * TGT_CHUNK
    return pltpu.make_async_copy(
        tbuf.at[c % 4], out_hbm.at[pl.ds(off, TGT_CHUNK)], tssem.at[c % 4])

  tgt_gather(0).start()
  tgt_gather(1).start()
  for c in range(NTC):
    tgt_gather(c).wait()
    if c >= 2:
      tgt_store(c - 2).wait()
    tgt_store(c).start()
    if c + 2 < NTC:
      tgt_gather(c + 2).start()
  tgt_store(NTC - 2).wait()
  tgt_store(NTC - 1).wait()


def _sc_cate_kernel(cate_hist_hbm, tcate_hbm, cate_tab_hbm,
                    tc_out, cnt_out,
                    ch_idx, cbuf, tidx_c, tbuf,
                    isem, ssem, tgsem, tssem):
  cid = lax.axis_index("c")
  sid = lax.axis_index("s")
  wid = sid * NC + cid
  base = wid * ROWS_PER_W

  pltpu.sync_copy(tcate_hbm.at[pl.ds(base, ROWS_PER_W)], tidx_c)
  _tgt_pipeline(cate_tab_hbm, tidx_c, tc_out, base, tbuf, tgsem, tssem)

  ones = jnp.ones((LANES,), jnp.float32)
  lastmask = lax.iota(jnp.int32, LANES) >= (2 * LANES - (L % (2 * LANES)))
  zeros = jnp.zeros((LANES,), jnp.float32)

  def idx_copy(k, p):
    b0 = base + k * G
    return pltpu.make_async_copy(
        cate_hist_hbm.at[pl.ds(b0, G)], ch_idx.at[p], isem.at[p])

  def store_copy(k, p):
    b0 = base + k * G
    return pltpu.make_async_copy(
        cbuf.at[p], cnt_out.at[pl.ds(b0, G)], ssem.at[p])

  for p in range(2):
    c = idx_copy(p, p)
    c.start()
    c.wait()

  def loop_body(jj, carry):
    for m in range(2):
      p = m
      k = 2 * jj + m
      @pl.when(jj >= 1)
      def _():
        store_copy(k, p).wait()
      @pl.when(jj <= (NGRP // 2 - 2))
      def _():
        idx_copy(k + 2, p).start()
      for g in range(G):
        for c in range(NCATE // LANES):
          cbuf[p, g, pl.ds(c * LANES, LANES)] = zeros
        gsplat = jnp.full((LANES,), g, jnp.int32)
        for c in range(L // LANES):
          idxv = ch_idx[p, g, pl.ds(c * LANES, LANES)]
          plsc.addupdate_scatter(cbuf.at[p], [gsplat, idxv], ones)
        idxv = ch_idx[p, g, pl.ds(L - LANES, LANES)]
        plsc.addupdate_scatter(cbuf.at[p], [gsplat, idxv], ones,
                               mask=lastmask)
      store_copy(k, p).start()
      @pl.when(jj <= (NGRP // 2 - 2))
      def _():
        idx_copy(k + 2, p).wait()
    return carry

  lax.fori_loop(0, NGRP // 2, loop_body, 0)

  for p in range(2):
    store_copy(NGRP - 2 + p, p).wait()


def _sc_item_kernel(item_hist_hbm, titem_hbm, item_tab_hbm,
                    ti_out, ihs_out,
                    ih_idx, ibuf, obuf_i, tidx_i, tbuf,
                    gsem, isem, ssem, tgsem, tssem):
  cid = lax.axis_index("c")
  sid = lax.axis_index("s")
  wid = sid * NC + cid
  base = wid * ROWS_PER_W

  pltpu.sync_copy(titem_hbm.at[pl.ds(base, ROWS_PER_W)], tidx_i)
  _tgt_pipeline(item_tab_hbm, tidx_i, ti_out, base, tbuf, tgsem, tssem)

  zeros = jnp.zeros((LANES,), jnp.float32)

  def idx_copy(k, p):
    b0 = base + k * G
    return pltpu.make_async_copy(
        item_hist_hbm.at[pl.ds(b0, G)], ih_idx.at[p], isem.at[p])

  def gather_copies(p):
    cps = []
    for g in range(G):
      for h in range(2):
        cps.append(pltpu.make_async_copy(
            item_tab_hbm.at[ih_idx.at[p, g, h]], ibuf.at[p, g, h],
            gsem.at[p]))
    return cps

  def store_copy(k, p):
    b0 = base + k * G
    return pltpu.make_async_copy(
        obuf_i.at[p], ihs_out.at[pl.ds(b0, G)], ssem.at[p])

  for p in range(2):
    c = idx_copy(p, p)
    c.start()
    c.wait()
    for c in gather_copies(p):
      c.start()

  def loop_body(jj, carry):
    for m in range(2):
      p = m
      k = 2 * jj + m
      for c in gather_copies(p):
        c.wait()
      @pl.when(jj >= 1)
      def _():
        store_copy(k, p).wait()
      @pl.when(jj <= (NGRP // 2 - 2))
      def _():
        idx_copy(k + 2, p).start()
      for g in range(G):
        def red_body(i, accs):
          a0, a1, a2, a3 = accs
          for h in range(2):
            for u in range(4):
              r = i * 4 + u
              a0 = a0 + ibuf[p, g, h, r, pl.ds(0, LANES)]
              a1 = a1 + ibuf[p, g, h, r, pl.ds(16, LANES)]
              a2 = a2 + ibuf[p, g, h, r, pl.ds(32, LANES)]
              a3 = a3 + ibuf[p, g, h, r, pl.ds(48, LANES)]
          return a0, a1, a2, a3

        accs = lax.fori_loop(0, HALF // 4, red_body, (zeros,) * 4)
        for d in range(4):
          obuf_i[p, g, pl.ds(d * LANES, LANES)] = accs[d]
      store_copy(k, p).start()
      @pl.when(jj <= (NGRP // 2 - 2))
      def _():
        idx_copy(k + 2, p).wait()
        for c in gather_copies(p):
          c.start()
    return carry

  lax.fori_loop(0, NGRP // 2, loop_body, 0)

  for p in range(2):
    store_copy(NGRP - 2 + p, p).wait()


def _sc_mesh():
  return plsc.VectorSubcoreMesh(core_axis_name="c", subcore_axis_name="s",
                                num_cores=NC, num_subcores=NS)


_SC_PARAMS = dict(
    compiler_params=pltpu.CompilerParams(
        use_tc_tiling_on_sc=False, needs_layout_passes=False))


def _sc_cate(cate_hist, tcate, cate_tab):
  f32 = jnp.float32
  i32 = jnp.int32
  out_type = (
      jax.ShapeDtypeStruct((B, D), f32),
      jax.ShapeDtypeStruct((B, NCATE), f32),
  )
  scratch = [
      pltpu.VMEM((2, G, L), i32),
      pltpu.VMEM((2, G, NCATE), f32),
      pltpu.VMEM((ROWS_PER_W,), i32),
      pltpu.VMEM((4, TGT_CHUNK, D), f32),
      pltpu.SemaphoreType.DMA((2,)),
      pltpu.SemaphoreType.DMA((2,)),
      pltpu.SemaphoreType.DMA((4,)),
      pltpu.SemaphoreType.DMA((4,)),
  ]
  fn = pl.kernel(_sc_cate_kernel, out_type=out_type, mesh=_sc_mesh(),
                 scratch_types=scratch, **_SC_PARAMS)
  return fn(cate_hist, tcate, cate_tab)


def _sc_item(item_hist, titem, item_tab):
  f32 = jnp.float32
  i32 = jnp.int32
  out_type = (
      jax.ShapeDtypeStruct((B, D), f32),
      jax.ShapeDtypeStruct((B, D), f32),
  )
  scratch = [
      pltpu.VMEM((2, G, 2, HALF), i32),
      pltpu.VMEM((2, G, 2, HALF, D), f32),
      pltpu.VMEM((2, G, D), f32),
      pltpu.VMEM((ROWS_PER_W,), i32),
      pltpu.VMEM((4, TGT_CHUNK, D), f32),
      pltpu.SemaphoreType.DMA((2,)),
      pltpu.SemaphoreType.DMA((2,)),
      pltpu.SemaphoreType.DMA((2,)),
      pltpu.SemaphoreType.DMA((4,)),
      pltpu.SemaphoreType.DMA((4,)),
  ]
  fn = pl.kernel(_sc_item_kernel, out_type=out_type, mesh=_sc_mesh(),
                 scratch_types=scratch, **_SC_PARAMS)
  return fn(item_hist, titem, item_tab)


BS = 1024


def _mlp_kernel(ti, tc, ihs, cnt, ctab, w1, b1, a1, w2, b2, a2, w3, b3, out):
  f32 = jnp.float32
  chs = jnp.dot(cnt[...], ctab[...], preferred_element_type=f32)
  x = jnp.concatenate([ti[...], tc[...], ihs[...], chs], axis=1)
  h1 = jnp.dot(x, w1[...], preferred_element_type=f32) + b1[...]
  h1 = jnp.maximum(h1, 0.0) + a1[...] * jnp.minimum(h1, 0.0)
  h2 = jnp.dot(h1, w2[...], preferred_element_type=f32) + b2[...]
  h2 = jnp.maximum(h2, 0.0) + a2[...] * jnp.minimum(h2, 0.0)
  z = jnp.dot(h2, w3[...], preferred_element_type=f32) + b3[...]
  m = jnp.max(z, axis=-1, keepdims=True)
  e = jnp.exp(z - m)
  out[...] = e / jnp.sum(e, axis=-1, keepdims=True) + 1e-8


def _mlp(ti, tc, ihs, cnt, ctab, w1, b1, a1, w2, b2, a2, w3, b3):
  n1, n2 = w2.shape
  n3 = w3.shape[1]
  row = lambda i: (i, 0)
  full = lambda i: (0, 0)
  bspec = lambda w: pl.BlockSpec((BS, w), row)
  wspec = lambda s: pl.BlockSpec(s, full)
  return pl.pallas_call(
      _mlp_kernel,
      grid=(B // BS,),
      in_specs=[
          bspec(D), bspec(D), bspec(D), bspec(NCATE),
          wspec((NCATE, D)),
          wspec((4 * D, n1)), wspec((1, n1)), wspec((1, n1)),
          wspec((n1, n2)), wspec((1, n2)), wspec((1, n2)),
          wspec((n2, n3)), wspec((1, n3)),
      ],
      out_specs=pl.BlockSpec((BS, n3), row),
      out_shape=jax.ShapeDtypeStruct((B, n3), jnp.float32),
  )(ti, tc, ihs, cnt, ctab, w1, b1, a1, w2, b2, a2, w3, b3)


def kernel(item_history, cate_history, targetitem, targetcate, item_lookup,
           cate_lookup, gamma, beta, W1, b1, a1, W2, b2, a2, W3, b3):
  tc, cnt = _sc_cate(cate_history, targetcate, cate_lookup)
  ih = item_history.reshape(B, 2, HALF)
  ti, ihs = _sc_item(ih, targetitem, item_lookup)
  ctab_p = jnp.pad(cate_lookup, ((0, NCATE - cate_lookup.shape[0]), (0, 0)))
  scale = gamma * (1.0 / jnp.sqrt(1.0 + 1e-3))
  w1 = W1 * scale[:, None]
  b1e = (b1 + beta @ W1).reshape(1, -1)
  out = _mlp(ti, tc, ihs, cnt, ctab_p, w1, b1e, a1.reshape(1, -1),
             W2, b2.reshape(1, -1), a2.reshape(1, -1),
             W3, b3.reshape(1, -1))
  return out
